# baseline (speedup 1.0000x reference)
import functools

import numpy as np
import jax
import jax.numpy as jnp
from jax import lax
from jax.experimental import pallas as pl
from jax.experimental.pallas import tpu as pltpu
from jax.experimental.pallas import tpu_sc as plsc

_N = 4096
_n = 8192
_m = 4
_sigma = 2.0
_BX = 4
_BF = 2
_M = 65536

_NW = 32
_CHUNKS = _NW // _BX
_CHUNK = _M // _CHUNKS
_VECS = _CHUNK // 16
_DEG = 12


def _window_poly_coeffs():
    b = (2.0 - 1.0 / _sigma) * np.pi
    fr = (np.cos(np.pi * (np.arange(400) + 0.5) / 400) + 1.0) / 2.0
    u = 2.0 * fr - 1.0
    from numpy.polynomial import chebyshev as C
    evens, odds = [], []
    for j in range(4):
        t = _m - j - fr
        s = _m * _m - t * t
        arg = np.sqrt(s)
        y = np.sinh(b * arg) / (arg * np.pi)
        c = C.chebfit(u, y, _DEG)
        p = C.cheb2poly(c)
        if len(p) < _DEG + 1:
            p = np.concatenate([p, np.zeros(_DEG + 1 - len(p))])
        sgn = (-1.0) ** j
        evens.append(sgn * p[0::2])
        odds.append(sgn * p[1::2])
    return np.asarray(evens, np.float64), np.asarray(odds, np.float64)


_EV, _OD = _window_poly_coeffs()


def _phi_hat_np():
    b = (2.0 - 1.0 / _sigma) * np.pi
    inds = np.arange(-(_N // 2), _N // 2, dtype=np.float64)
    return np.i0(_m * np.sqrt(b * b - (2.0 * np.pi * inds / _n) ** 2))


_PHI_HAT = np.asarray(_phi_hat_np(), np.float32)


def _horner(coeffs, v):
    acc = jnp.float32(coeffs[-1]) * v + jnp.float32(coeffs[-2])
    for k in range(len(coeffs) - 3, -1, -1):
        acc = acc * v + jnp.float32(coeffs[k])
    return acc


def _tec_kernel(gre_h, gim_h, x_h, out_h,
                t0r, t0i, t1r, t1i, xv, o0, o1):
    cid = lax.axis_index("c")
    sid = lax.axis_index("s")
    wid = sid * 2 + cid
    bx = wid // _CHUNKS
    pltpu.sync_copy(gre_h.at[bx * _BF + 0], t0r)
    pltpu.sync_copy(gim_h.at[bx * _BF + 0], t0i)
    pltpu.sync_copy(gre_h.at[bx * _BF + 1], t1r)
    pltpu.sync_copy(gim_h.at[bx * _BF + 1], t1i)
    pltpu.sync_copy(x_h.at[wid], xv)

    lane2 = 2 * lax.iota(jnp.int32, 16)

    @plsc.parallel_loop(0, _CHUNK, 16, unroll=2)
    def body(bs):
        xv16 = xv[pl.ds(bs, 16)]
        y = xv16 * jnp.float32(_n)
        ti = y.astype(jnp.int32)
        tf = ti.astype(jnp.float32)
        ci = jnp.where(y > tf, ti + 1, ti)
        cf = ci.astype(jnp.float32)
        frac = cf - y
        uu = 2.0 * frac - 1.0
        vv = uu * uu
        bidx = ci + jnp.int32(_n - _m)
        sf = 1.0 - 2.0 * jnp.bitwise_and(ci, 1).astype(jnp.float32)
        a0r = jnp.zeros((16,), jnp.float32)
        a0i = jnp.zeros((16,), jnp.float32)
        a1r = jnp.zeros((16,), jnp.float32)
        a1i = jnp.zeros((16,), jnp.float32)
        for j in range(4):
            e = _horner(_EV[j], vv)
            o = _horner(_OD[j], vv)
            uo = uu * o
            wlo = e + uo
            whi = uo - e
            if j == 0:
                wlo = jnp.where(frac > 0.0, wlo, 0.0)
            for jj, w in ((j, wlo), (7 - j, whi)):
                idx = jnp.bitwise_and(bidx + jnp.int32(jj), jnp.int32(_n - 1))
                a0r = a0r + w * plsc.load_gather(t0r, [idx])
                a0i = a0i + w * plsc.load_gather(t0i, [idx])
                a1r = a1r + w * plsc.load_gather(t1r, [idx])
                a1i = a1i + w * plsc.load_gather(t1i, [idx])
        oe = lane2 + (2 * bs)
        oo = oe + 1
        plsc.store_scatter(o0, [oe], sf * a0r)
        plsc.store_scatter(o0, [oo], sf * a0i)
        plsc.store_scatter(o1, [oe], sf * a1r)
        plsc.store_scatter(o1, [oo], sf * a1i)

    ch = wid % _CHUNKS
    pltpu.sync_copy(o0, out_h.at[(bx * _BF + 0) * _CHUNKS + ch])
    pltpu.sync_copy(o1, out_h.at[(bx * _BF + 1) * _CHUNKS + ch])


@jax.jit
def kernel(x, f_hat):
    phi_hat = jnp.asarray(_PHI_HAT)
    g_hat = f_hat / phi_hat
    pad = (_n - _N) // 2
    g_hat = jnp.pad(g_hat, ((0, 0), (0, 0), (pad, pad)))
    g = jnp.fft.fft(g_hat)
    gre = jnp.real(g).reshape(_BX * _BF, _n).astype(jnp.float32)
    gim = jnp.imag(g).reshape(_BX * _BF, _n).astype(jnp.float32)
    xr = x.reshape(_NW, _CHUNK)

    sc_call = pl.kernel(
        _tec_kernel,
        out_type=jax.ShapeDtypeStruct(
            (_BX * _BF * _CHUNKS, 2 * _CHUNK), jnp.float32),
        mesh=plsc.VectorSubcoreMesh(core_axis_name="c", subcore_axis_name="s"),
        compiler_params=pltpu.CompilerParams(needs_layout_passes=False),
        scratch_types=[
            pltpu.VMEM((_n,), jnp.float32),
            pltpu.VMEM((_n,), jnp.float32),
            pltpu.VMEM((_n,), jnp.float32),
            pltpu.VMEM((_n,), jnp.float32),
            pltpu.VMEM((_CHUNK,), jnp.float32),
            pltpu.VMEM((2 * _CHUNK,), jnp.float32),
            pltpu.VMEM((2 * _CHUNK,), jnp.float32),
        ],
    )
    pairs = sc_call(gre, gim, xr).reshape(_BX, _BF, _M, 2)
    return lax.complex(pairs[..., 0], pairs[..., 1])


# speedup vs baseline: 8.0292x; 8.0292x over previous
import functools

import numpy as np
import jax
import jax.numpy as jnp
from jax import lax
from jax.experimental import pallas as pl
from jax.experimental.pallas import tpu as pltpu
from jax.experimental.pallas import tpu_sc as plsc

_N = 4096
_n = 8192
_m = 4
_sigma = 2.0
_BX = 4
_BF = 2
_M = 65536

_NW = 32
_CHUNKS = _NW // _BX
_CHUNK = _M // _CHUNKS
_VECS = _CHUNK // 16
_DEG = 12


def _window_poly_coeffs():
    b = (2.0 - 1.0 / _sigma) * np.pi
    fr = (np.cos(np.pi * (np.arange(400) + 0.5) / 400) + 1.0) / 2.0
    u = 2.0 * fr - 1.0
    from numpy.polynomial import chebyshev as C
    evens, odds = [], []
    for j in range(4):
        t = _m - j - fr
        s = _m * _m - t * t
        arg = np.sqrt(s)
        y = np.sinh(b * arg) / (arg * np.pi)
        c = C.chebfit(u, y, _DEG)
        p = C.cheb2poly(c)
        if len(p) < _DEG + 1:
            p = np.concatenate([p, np.zeros(_DEG + 1 - len(p))])
        sgn = (-1.0) ** j
        evens.append(sgn * p[0::2])
        odds.append(sgn * p[1::2])
    return np.asarray(evens, np.float64), np.asarray(odds, np.float64)


_EV, _OD = _window_poly_coeffs()


def _phi_hat_np():
    b = (2.0 - 1.0 / _sigma) * np.pi
    inds = np.arange(-(_N // 2), _N // 2, dtype=np.float64)
    return np.i0(_m * np.sqrt(b * b - (2.0 * np.pi * inds / _n) ** 2))


_PHI_HAT = np.asarray(_phi_hat_np(), np.float32)


def _horner(coeffs, v):
    acc = jnp.float32(coeffs[-1]) * v + jnp.float32(coeffs[-2])
    for k in range(len(coeffs) - 3, -1, -1):
        acc = acc * v + jnp.float32(coeffs[k])
    return acc


def _tec_kernel(gre_h, gim_h, x_h, ore_h, oim_h,
                t0r, t0i, t1r, t1i, xv, o0r, o0i, o1r, o1i):
    cid = lax.axis_index("c")
    sid = lax.axis_index("s")
    wid = sid * 2 + cid
    bx = wid // _CHUNKS
    pltpu.sync_copy(gre_h.at[bx * _BF + 0], t0r)
    pltpu.sync_copy(gim_h.at[bx * _BF + 0], t0i)
    pltpu.sync_copy(gre_h.at[bx * _BF + 1], t1r)
    pltpu.sync_copy(gim_h.at[bx * _BF + 1], t1i)
    pltpu.sync_copy(x_h.at[wid], xv)

    @plsc.parallel_loop(0, _CHUNK, 16, unroll=2)
    def body(bs):
        xv16 = xv[pl.ds(bs, 16)]
        y = xv16 * jnp.float32(_n)
        ti = y.astype(jnp.int32)
        tf = ti.astype(jnp.float32)
        ci = jnp.where(y > tf, ti + 1, ti)
        cf = ci.astype(jnp.float32)
        frac = cf - y
        uu = 2.0 * frac - 1.0
        vv = uu * uu
        bidx = ci + jnp.int32(_n - _m)
        sf = 1.0 - 2.0 * jnp.bitwise_and(ci, 1).astype(jnp.float32)
        a0r = jnp.zeros((16,), jnp.float32)
        a0i = jnp.zeros((16,), jnp.float32)
        a1r = jnp.zeros((16,), jnp.float32)
        a1i = jnp.zeros((16,), jnp.float32)
        for j in range(4):
            e = _horner(_EV[j], vv)
            o = _horner(_OD[j], vv)
            uo = uu * o
            wlo = e + uo
            whi = uo - e
            if j == 0:
                wlo = jnp.where(frac > 0.0, wlo, 0.0)
            for jj, w in ((j, wlo), (7 - j, whi)):
                idx = jnp.bitwise_and(bidx + jnp.int32(jj), jnp.int32(_n - 1))
                a0r = a0r + w * plsc.load_gather(t0r, [idx])
                a0i = a0i + w * plsc.load_gather(t0i, [idx])
                a1r = a1r + w * plsc.load_gather(t1r, [idx])
                a1i = a1i + w * plsc.load_gather(t1i, [idx])
        o0r[pl.ds(bs, 16)] = sf * a0r
        o0i[pl.ds(bs, 16)] = sf * a0i
        o1r[pl.ds(bs, 16)] = sf * a1r
        o1i[pl.ds(bs, 16)] = sf * a1i

    ch = wid % _CHUNKS
    pltpu.sync_copy(o0r, ore_h.at[bx, 0, pl.ds(ch * _CHUNK, _CHUNK)])
    pltpu.sync_copy(o0i, oim_h.at[bx, 0, pl.ds(ch * _CHUNK, _CHUNK)])
    pltpu.sync_copy(o1r, ore_h.at[bx, 1, pl.ds(ch * _CHUNK, _CHUNK)])
    pltpu.sync_copy(o1i, oim_h.at[bx, 1, pl.ds(ch * _CHUNK, _CHUNK)])


@jax.jit
def kernel(x, f_hat):
    phi_hat = jnp.asarray(_PHI_HAT)
    g_hat = f_hat / phi_hat
    pad = (_n - _N) // 2
    g_hat = jnp.pad(g_hat, ((0, 0), (0, 0), (pad, pad)))
    g = jnp.fft.fft(g_hat)
    gre = jnp.real(g).reshape(_BX * _BF, _n).astype(jnp.float32)
    gim = jnp.imag(g).reshape(_BX * _BF, _n).astype(jnp.float32)
    xr = x.reshape(_NW, _CHUNK)

    sc_call = pl.kernel(
        _tec_kernel,
        out_type=[
            jax.ShapeDtypeStruct((_BX, _BF, _M), jnp.float32),
            jax.ShapeDtypeStruct((_BX, _BF, _M), jnp.float32),
        ],
        mesh=plsc.VectorSubcoreMesh(core_axis_name="c", subcore_axis_name="s"),
        compiler_params=pltpu.CompilerParams(needs_layout_passes=False),
        scratch_types=[
            pltpu.VMEM((_n,), jnp.float32),
            pltpu.VMEM((_n,), jnp.float32),
            pltpu.VMEM((_n,), jnp.float32),
            pltpu.VMEM((_n,), jnp.float32),
            pltpu.VMEM((_CHUNK,), jnp.float32),
            pltpu.VMEM((_CHUNK,), jnp.float32),
            pltpu.VMEM((_CHUNK,), jnp.float32),
            pltpu.VMEM((_CHUNK,), jnp.float32),
            pltpu.VMEM((_CHUNK,), jnp.float32),
        ],
    )
    ore, oim = sc_call(gre, gim, xr)
    return (ore + 1j * oim).astype(jnp.complex64)


# speedup vs baseline: 8.0619x; 1.0041x over previous
import functools

import numpy as np
import jax
import jax.numpy as jnp
from jax import lax
from jax.experimental import pallas as pl
from jax.experimental.pallas import tpu as pltpu
from jax.experimental.pallas import tpu_sc as plsc

_N = 4096
_n = 8192
_m = 4
_sigma = 2.0
_BX = 4
_BF = 2
_M = 65536

_NW = 32
_CHUNKS = _NW // _BX
_CHUNK = _M // _CHUNKS
_VECS = _CHUNK // 16
_DEG = 10


def _window_poly_coeffs():
    b = (2.0 - 1.0 / _sigma) * np.pi
    fr = (np.cos(np.pi * (np.arange(400) + 0.5) / 400) + 1.0) / 2.0
    u = 2.0 * fr - 1.0
    from numpy.polynomial import chebyshev as C
    evens, odds = [], []
    for j in range(4):
        t = _m - j - fr
        s = _m * _m - t * t
        arg = np.sqrt(s)
        y = np.sinh(b * arg) / (arg * np.pi)
        c = C.chebfit(u, y, _DEG)
        p = C.cheb2poly(c)
        if len(p) < _DEG + 1:
            p = np.concatenate([p, np.zeros(_DEG + 1 - len(p))])
        sgn = (-1.0) ** j
        evens.append(sgn * p[0::2])
        odds.append(sgn * p[1::2])
    return np.asarray(evens, np.float64), np.asarray(odds, np.float64)


_EV, _OD = _window_poly_coeffs()


def _phi_hat_np():
    b = (2.0 - 1.0 / _sigma) * np.pi
    inds = np.arange(-(_N // 2), _N // 2, dtype=np.float64)
    return np.i0(_m * np.sqrt(b * b - (2.0 * np.pi * inds / _n) ** 2))


_PHI_HAT = np.asarray(_phi_hat_np(), np.float32)


def _horner(coeffs, v):
    acc = jnp.float32(coeffs[-1]) * v + jnp.float32(coeffs[-2])
    for k in range(len(coeffs) - 3, -1, -1):
        acc = acc * v + jnp.float32(coeffs[k])
    return acc


def _tec_kernel(gre_h, gim_h, x_h, ore_h, oim_h,
                t0r, t0i, t1r, t1i, xv, o0r, o0i, o1r, o1i):
    cid = lax.axis_index("c")
    sid = lax.axis_index("s")
    wid = sid * 2 + cid
    bx = wid // _CHUNKS
    pltpu.sync_copy(gre_h.at[bx * _BF + 0], t0r)
    pltpu.sync_copy(gim_h.at[bx * _BF + 0], t0i)
    pltpu.sync_copy(gre_h.at[bx * _BF + 1], t1r)
    pltpu.sync_copy(gim_h.at[bx * _BF + 1], t1i)
    pltpu.sync_copy(x_h.at[wid], xv)

    @plsc.parallel_loop(0, _CHUNK, 16, unroll=3)
    def body(bs):
        xv16 = xv[pl.ds(bs, 16)]
        y = xv16 * jnp.float32(_n)
        ti = y.astype(jnp.int32)
        tf = ti.astype(jnp.float32)
        ci = jnp.where(y > tf, ti + 1, ti)
        cf = ci.astype(jnp.float32)
        frac = cf - y
        uu = 2.0 * frac - 1.0
        vv = uu * uu
        bidx = ci + jnp.int32(_n - _m)
        sf = 1.0 - 2.0 * jnp.bitwise_and(ci, 1).astype(jnp.float32)
        a0r = jnp.zeros((16,), jnp.float32)
        a0i = jnp.zeros((16,), jnp.float32)
        a1r = jnp.zeros((16,), jnp.float32)
        a1i = jnp.zeros((16,), jnp.float32)
        for j in range(4):
            e = _horner(_EV[j], vv)
            o = _horner(_OD[j], vv)
            uo = uu * o
            wlo = e + uo
            whi = uo - e
            if j == 0:
                wlo = jnp.where(frac > 0.0, wlo, 0.0)
            for jj, w in ((j, wlo), (7 - j, whi)):
                idx = jnp.bitwise_and(bidx + jnp.int32(jj), jnp.int32(_n - 1))
                a0r = a0r + w * plsc.load_gather(t0r, [idx])
                a0i = a0i + w * plsc.load_gather(t0i, [idx])
                a1r = a1r + w * plsc.load_gather(t1r, [idx])
                a1i = a1i + w * plsc.load_gather(t1i, [idx])
        o0r[pl.ds(bs, 16)] = sf * a0r
        o0i[pl.ds(bs, 16)] = sf * a0i
        o1r[pl.ds(bs, 16)] = sf * a1r
        o1i[pl.ds(bs, 16)] = sf * a1i

    ch = wid % _CHUNKS
    pltpu.sync_copy(o0r, ore_h.at[bx, 0, pl.ds(ch * _CHUNK, _CHUNK)])
    pltpu.sync_copy(o0i, oim_h.at[bx, 0, pl.ds(ch * _CHUNK, _CHUNK)])
    pltpu.sync_copy(o1r, ore_h.at[bx, 1, pl.ds(ch * _CHUNK, _CHUNK)])
    pltpu.sync_copy(o1i, oim_h.at[bx, 1, pl.ds(ch * _CHUNK, _CHUNK)])


@jax.jit
def kernel(x, f_hat):
    phi_hat = jnp.asarray(_PHI_HAT)
    g_hat = f_hat / phi_hat
    pad = (_n - _N) // 2
    g_hat = jnp.pad(g_hat, ((0, 0), (0, 0), (pad, pad)))
    g = jnp.fft.fft(g_hat)
    gre = jnp.real(g).reshape(_BX * _BF, _n).astype(jnp.float32)
    gim = jnp.imag(g).reshape(_BX * _BF, _n).astype(jnp.float32)
    xr = x.reshape(_NW, _CHUNK)

    sc_call = pl.kernel(
        _tec_kernel,
        out_type=[
            jax.ShapeDtypeStruct((_BX, _BF, _M), jnp.float32),
            jax.ShapeDtypeStruct((_BX, _BF, _M), jnp.float32),
        ],
        mesh=plsc.VectorSubcoreMesh(core_axis_name="c", subcore_axis_name="s"),
        compiler_params=pltpu.CompilerParams(needs_layout_passes=False),
        scratch_types=[
            pltpu.VMEM((_n,), jnp.float32),
            pltpu.VMEM((_n,), jnp.float32),
            pltpu.VMEM((_n,), jnp.float32),
            pltpu.VMEM((_n,), jnp.float32),
            pltpu.VMEM((_CHUNK,), jnp.float32),
            pltpu.VMEM((_CHUNK,), jnp.float32),
            pltpu.VMEM((_CHUNK,), jnp.float32),
            pltpu.VMEM((_CHUNK,), jnp.float32),
            pltpu.VMEM((_CHUNK,), jnp.float32),
        ],
    )
    ore, oim = sc_call(gre, gim, xr)
    return (ore + 1j * oim).astype(jnp.complex64)


# speedup vs baseline: 8.2255x; 1.0203x over previous
import functools

import numpy as np
import jax
import jax.numpy as jnp
from jax import lax
from jax.experimental import pallas as pl
from jax.experimental.pallas import tpu as pltpu
from jax.experimental.pallas import tpu_sc as plsc

_N = 4096
_n = 8192
_m = 4
_sigma = 2.0
_BX = 4
_BF = 2
_M = 65536

_NW = 32
_CHUNKS = _NW // _BX
_CHUNK = _M // _CHUNKS
_VECS = _CHUNK // 16
_DEG = 8


def _window_poly_coeffs():
    b = (2.0 - 1.0 / _sigma) * np.pi
    fr = (np.cos(np.pi * (np.arange(400) + 0.5) / 400) + 1.0) / 2.0
    u = 2.0 * fr - 1.0
    from numpy.polynomial import chebyshev as C
    evens, odds = [], []
    for j in range(4):
        t = _m - j - fr
        s = _m * _m - t * t
        arg = np.sqrt(s)
        y = np.sinh(b * arg) / (arg * np.pi)
        c = C.chebfit(u, y, _DEG)
        p = C.cheb2poly(c)
        if len(p) < _DEG + 1:
            p = np.concatenate([p, np.zeros(_DEG + 1 - len(p))])
        sgn = (-1.0) ** j
        evens.append(sgn * p[0::2])
        odds.append(sgn * p[1::2])
    return np.asarray(evens, np.float64), np.asarray(odds, np.float64)


_EV, _OD = _window_poly_coeffs()


def _phi_hat_np():
    b = (2.0 - 1.0 / _sigma) * np.pi
    inds = np.arange(-(_N // 2), _N // 2, dtype=np.float64)
    return np.i0(_m * np.sqrt(b * b - (2.0 * np.pi * inds / _n) ** 2))


_PHI_HAT = np.asarray(_phi_hat_np(), np.float32)


def _horner(coeffs, v):
    acc = jnp.float32(coeffs[-1]) * v + jnp.float32(coeffs[-2])
    for k in range(len(coeffs) - 3, -1, -1):
        acc = acc * v + jnp.float32(coeffs[k])
    return acc


def _tec_kernel(gre_h, gim_h, x_h, ore_h, oim_h,
                t0r, t0i, t1r, t1i, xv, o0r, o0i, o1r, o1i):
    cid = lax.axis_index("c")
    sid = lax.axis_index("s")
    wid = sid * 2 + cid
    bx = wid // _CHUNKS
    pltpu.sync_copy(gre_h.at[bx * _BF + 0], t0r)
    pltpu.sync_copy(gim_h.at[bx * _BF + 0], t0i)
    pltpu.sync_copy(gre_h.at[bx * _BF + 1], t1r)
    pltpu.sync_copy(gim_h.at[bx * _BF + 1], t1i)
    pltpu.sync_copy(x_h.at[wid], xv)

    @plsc.parallel_loop(0, _CHUNK, 16, unroll=3)
    def body(bs):
        xv16 = xv[pl.ds(bs, 16)]
        y = xv16 * jnp.float32(_n)
        ti = y.astype(jnp.int32)
        tf = ti.astype(jnp.float32)
        ci = jnp.where(y > tf, ti + 1, ti)
        cf = ci.astype(jnp.float32)
        frac = cf - y
        uu = 2.0 * frac - 1.0
        vv = uu * uu
        bidx = ci + jnp.int32(_n - _m)
        sf = 1.0 - 2.0 * jnp.bitwise_and(ci, 1).astype(jnp.float32)
        a0r = jnp.zeros((16,), jnp.float32)
        a0i = jnp.zeros((16,), jnp.float32)
        a1r = jnp.zeros((16,), jnp.float32)
        a1i = jnp.zeros((16,), jnp.float32)
        for j in range(4):
            e = _horner(_EV[j], vv)
            o = _horner(_OD[j], vv)
            uo = uu * o
            wlo = e + uo
            whi = uo - e
            if j == 0:
                wlo = jnp.where(frac > 0.0, wlo, 0.0)
            for jj, w in ((j, wlo), (7 - j, whi)):
                idx = jnp.bitwise_and(bidx + jnp.int32(jj), jnp.int32(_n - 1))
                a0r = a0r + w * plsc.load_gather(t0r, [idx])
                a0i = a0i + w * plsc.load_gather(t0i, [idx])
                a1r = a1r + w * plsc.load_gather(t1r, [idx])
                a1i = a1i + w * plsc.load_gather(t1i, [idx])
        o0r[pl.ds(bs, 16)] = sf * a0r
        o0i[pl.ds(bs, 16)] = sf * a0i
        o1r[pl.ds(bs, 16)] = sf * a1r
        o1i[pl.ds(bs, 16)] = sf * a1i

    ch = wid % _CHUNKS
    pltpu.sync_copy(o0r, ore_h.at[bx, 0, pl.ds(ch * _CHUNK, _CHUNK)])
    pltpu.sync_copy(o0i, oim_h.at[bx, 0, pl.ds(ch * _CHUNK, _CHUNK)])
    pltpu.sync_copy(o1r, ore_h.at[bx, 1, pl.ds(ch * _CHUNK, _CHUNK)])
    pltpu.sync_copy(o1i, oim_h.at[bx, 1, pl.ds(ch * _CHUNK, _CHUNK)])


@jax.jit
def kernel(x, f_hat):
    phi_hat = jnp.asarray(_PHI_HAT)
    g_hat = f_hat / phi_hat
    pad = (_n - _N) // 2
    g_hat = jnp.pad(g_hat, ((0, 0), (0, 0), (pad, pad)))
    g = jnp.fft.fft(g_hat)
    gre = jnp.real(g).reshape(_BX * _BF, _n).astype(jnp.float32)
    gim = jnp.imag(g).reshape(_BX * _BF, _n).astype(jnp.float32)
    xr = x.reshape(_NW, _CHUNK)

    sc_call = pl.kernel(
        _tec_kernel,
        out_type=[
            jax.ShapeDtypeStruct((_BX, _BF, _M), jnp.float32),
            jax.ShapeDtypeStruct((_BX, _BF, _M), jnp.float32),
        ],
        mesh=plsc.VectorSubcoreMesh(core_axis_name="c", subcore_axis_name="s"),
        compiler_params=pltpu.CompilerParams(needs_layout_passes=False),
        scratch_types=[
            pltpu.VMEM((_n,), jnp.float32),
            pltpu.VMEM((_n,), jnp.float32),
            pltpu.VMEM((_n,), jnp.float32),
            pltpu.VMEM((_n,), jnp.float32),
            pltpu.VMEM((_CHUNK,), jnp.float32),
            pltpu.VMEM((_CHUNK,), jnp.float32),
            pltpu.VMEM((_CHUNK,), jnp.float32),
            pltpu.VMEM((_CHUNK,), jnp.float32),
            pltpu.VMEM((_CHUNK,), jnp.float32),
        ],
    )
    ore, oim = sc_call(gre, gim, xr)
    return (ore + 1j * oim).astype(jnp.complex64)
